# Initial kernel scaffold; baseline (speedup 1.0000x reference)
import jax
import jax.numpy as jnp
from jax.experimental import pallas as pl


def kernel(x, gene_indices, tf_indices, weights):
    raise NotImplementedError("write your pallas kernel here")


# speedup vs baseline: 1.4102x; 1.4102x over previous
import functools

import jax
import jax.numpy as jnp
from jax import lax
from jax.experimental import pallas as pl
from jax.experimental.pallas import tpu as pltpu
from jax.experimental.pallas import tpu_sc as plsc

N_GENES_K = 20000
N_TFS_K = 1500
TF_PAD = 1536
BG = 4
BW = 64
CHUNK = 128
SUP = 8
N_SUP = 25
ROWS_PER_TILE = SUP * N_SUP
CONN_PAD = 8 * ROWS_PER_TILE * CHUNK
RED_ROWS = TF_PAD // 8


def _sc_body(xt_hbm, gene_hbm, tf_hbm, w_hbm, out_hbm, part_hbm,
             gene2, tf2, w2, idx2, gbuf, acc, gsem):
    c = lax.axis_index("c")
    s = lax.axis_index("s")
    g = c * 2 + s // 8
    tig = s % 8

    def zero_body(i, _):
        r = i // (BW // 16)
        sl = pl.ds((i % (BW // 16)) * 16, 16)
        acc[r, sl] = jnp.zeros((16,), jnp.float32)
        return 0
    lax.fori_loop(0, TF_PAD * BW // 16, zero_body, 0)

    gene_base = g * N_GENES_K

    def sup_body(si, _):
        row0 = tig * ROWS_PER_TILE + si * SUP
        pltpu.sync_copy(gene_hbm.at[pl.ds(row0, SUP)], gene2)
        pltpu.sync_copy(tf_hbm.at[pl.ds(row0, SUP)], tf2)
        pltpu.sync_copy(w_hbm.at[pl.ds(row0, SUP)], w2)

        def idx_body(r, _):
            for k in range(CHUNK // 16):
                idx2[r, pl.ds(k * 16, 16)] = (
                    gene2[r, pl.ds(k * 16, 16)] + gene_base)
            return 0
        lax.fori_loop(0, SUP, idx_body, 0)

        def chunk_body(r, _):
            pltpu.async_copy(xt_hbm.at[idx2.at[r]], gbuf, gsem).wait()

            def conn_body(cc, _):
                tf_v = tf2[r, pl.ds(cc * 16, 16)]
                w_v = w2[r, pl.ds(cc * 16, 16)]
                for l in range(16):
                    t = tf_v[l]
                    w = w_v[l]
                    j = cc * 16 + l
                    for k in range(BW // 16):
                        sl = pl.ds(k * 16, 16)
                        acc[t, sl] = acc[t, sl] + gbuf[j, sl] * w
                return 0
            lax.fori_loop(0, CHUNK // 16, conn_body, 0)
            return 0
        lax.fori_loop(0, SUP, chunk_body, 0)
        return 0
    lax.fori_loop(0, N_SUP, sup_body, 0)

    wid = c * 16 + s
    pltpu.sync_copy(acc, part_hbm.at[wid])
    plsc.subcore_barrier()

    r0 = tig * RED_ROWS
    wid0 = c * 16 + (g % 2) * 8
    pltpu.sync_copy(part_hbm.at[wid0, pl.ds(r0, RED_ROWS)],
                    acc.at[pl.ds(0, RED_ROWS)])

    def red_body(p, _):
        pltpu.sync_copy(part_hbm.at[wid0 + p, pl.ds(r0, RED_ROWS)],
                        acc.at[pl.ds(RED_ROWS, RED_ROWS)])

        def add16(i, _):
            dst_r = i // (BW // 16)
            kk = i % (BW // 16)
            sl = pl.ds(kk * 16, 16)
            acc[dst_r, sl] = acc[dst_r, sl] + acc[RED_ROWS + dst_r, sl]
            return 0
        lax.fori_loop(0, RED_ROWS * (BW // 16), add16, 0)
        return 0
    lax.fori_loop(1, 8, red_body, 0)

    pltpu.sync_copy(acc.at[pl.ds(0, RED_ROWS)],
                    out_hbm.at[g, pl.ds(r0, RED_ROWS)])


@jax.jit
def kernel(x, gene_indices, tf_indices, weights):
    b, n_genes = x.shape
    n_conn = gene_indices.shape[0]

    xt = x.reshape(BG, BW, n_genes).transpose(0, 2, 1).reshape(
        BG * n_genes, BW)

    pad = CONN_PAD - n_conn
    gene_h = jnp.pad(gene_indices, (0, pad)).reshape(-1, CHUNK)
    tf_h = jnp.pad(tf_indices, (0, pad)).reshape(-1, CHUNK)
    w_h = jnp.pad(weights, (0, pad)).reshape(-1, CHUNK)

    mesh = plsc.VectorSubcoreMesh(core_axis_name="c", subcore_axis_name="s")
    out, _ = pl.kernel(
        _sc_body,
        out_type=(
            jax.ShapeDtypeStruct((BG, TF_PAD, BW), jnp.float32),
            jax.ShapeDtypeStruct((32, TF_PAD, BW), jnp.float32),
        ),
        mesh=mesh,
        compiler_params=pltpu.CompilerParams(use_tc_tiling_on_sc=False),
        scratch_types=[
            pltpu.VMEM((SUP, CHUNK), jnp.int32),
            pltpu.VMEM((SUP, CHUNK), jnp.int32),
            pltpu.VMEM((SUP, CHUNK), jnp.float32),
            pltpu.VMEM((SUP, CHUNK), jnp.int32),
            pltpu.VMEM((CHUNK, BW), jnp.float32),
            pltpu.VMEM((TF_PAD, BW), jnp.float32),
            pltpu.SemaphoreType.DMA,
        ],
    )(xt, gene_h, tf_h, w_h)

    out = out[:, :N_TFS_K, :]
    return out.transpose(0, 2, 1).reshape(b, N_TFS_K)
